# Initial kernel scaffold; baseline (speedup 1.0000x reference)
import jax
import jax.numpy as jnp
from jax.experimental import pallas as pl


def kernel(x, conv1_w, conv1_b, conv2_w, conv2_b, fc1_w, fc1_b, fc2_w, fc2_b, gate_w, expert_w, expert_b, fc4_w, fc4_b):
    raise NotImplementedError("write your pallas kernel here")


# speedup vs baseline: 2.3257x; 2.3257x over previous
import functools

import jax
import jax.numpy as jnp
from jax.experimental import pallas as pl


def _toeplitz_conv1(conv1_w, parity):
    iw = jnp.arange(32)[:, None]
    j = jnp.arange(14)[None, :]
    kx = iw - (2 * j + parity)
    mask = (kx >= 0) & (kx < 5)
    g = conv1_w[:, :, :, jnp.clip(kx, 0, 4)]
    g = g * mask[None, None, None, :, :]
    return g.transpose(2, 1, 3, 0, 4).reshape(5, 96, 84)


def _toeplitz_conv2(conv2_w, parity):
    iw = jnp.arange(14)[:, None]
    j = jnp.arange(5)[None, :]
    kx = iw - (2 * j + parity)
    mask = (kx >= 0) & (kx < 5)
    g = conv2_w[:, :, :, jnp.clip(kx, 0, 4)]
    g = g * mask[None, None, None, :, :]
    return g.transpose(2, 1, 3, 0, 4).reshape(5, 84, 80)


def _net_kernel(nb, xt_ref, w1e_ref, w1o_ref, b1_ref, w2e_ref, w2o_ref,
                b2_ref, wfc1_ref, bfc1_ref, wfc2_ref, bfc2_ref, gd_ref,
                we_ref, be_ref, wf4_ref, bf4_ref, out_ref):
    f32 = jnp.float32
    x = xt_ref[...]

    acc_e = None
    acc_o = None
    for ky in range(5):
        a = x[:, ky:ky + 28, :].reshape(nb * 28, 96)
        pe = jnp.dot(a, w1e_ref[ky], preferred_element_type=f32)
        po = jnp.dot(a, w1o_ref[ky], preferred_element_type=f32)
        acc_e = pe if acc_e is None else acc_e + pe
        acc_o = po if acc_o is None else acc_o + po
    t = jnp.maximum(acc_e, acc_o) + b1_ref[...]
    t = t.reshape(nb, 14, 2, 84).max(axis=2)
    h1 = jnp.maximum(t, 0.0)

    acc_e = None
    acc_o = None
    for ky in range(5):
        a = h1[:, ky:ky + 10, :].reshape(nb * 10, 84)
        pe = jnp.dot(a, w2e_ref[ky], preferred_element_type=f32)
        po = jnp.dot(a, w2o_ref[ky], preferred_element_type=f32)
        acc_e = pe if acc_e is None else acc_e + pe
        acc_o = po if acc_o is None else acc_o + po
    t2 = jnp.maximum(acc_e, acc_o) + b2_ref[...]
    t2 = t2.reshape(nb, 5, 2, 80).max(axis=2)
    h2 = jnp.maximum(t2, 0.0)

    acc3 = None
    for y in range(5):
        p = jnp.dot(h2[:, y, :], wfc1_ref[y], preferred_element_type=f32)
        acc3 = p if acc3 is None else acc3 + p
    h3 = jnp.maximum(acc3 + bfc1_ref[...], 0.0)

    h4 = jnp.dot(h3, wfc2_ref[...], preferred_element_type=f32)
    h4 = jnp.maximum(h4 + bfc2_ref[...], 0.0)

    dlog = jnp.dot(h4, gd_ref[...], preferred_element_type=f32)
    gate_val = jax.nn.sigmoid(jnp.abs(dlog))
    sel = dlog > 0.0

    eo0 = jnp.dot(h4, we_ref[0], preferred_element_type=f32) + be_ref[0:1, :]
    eo1 = jnp.dot(h4, we_ref[1], preferred_element_type=f32) + be_ref[1:2, :]
    eo = jnp.where(sel, eo1, eo0) * gate_val

    out_ref[...] = jnp.dot(eo, wf4_ref[...], preferred_element_type=f32) \
        + bf4_ref[...]


def kernel(x, conv1_w, conv1_b, conv2_w, conv2_b, fc1_w, fc1_b, fc2_w,
           fc2_b, gate_w, expert_w, expert_b, fc4_w, fc4_b):
    b = x.shape[0]
    nb = 256 if b % 256 == 0 else b

    xt = x.transpose(0, 2, 1, 3).reshape(b, 32, 96)

    w1e = _toeplitz_conv1(conv1_w, 0)
    w1o = _toeplitz_conv1(conv1_w, 1)
    b1 = jnp.repeat(conv1_b, 14)[None, :]
    w2e = _toeplitz_conv2(conv2_w, 0)
    w2o = _toeplitz_conv2(conv2_w, 1)
    b2 = jnp.repeat(conv2_b, 5)[None, :]
    wfc1 = fc1_w.reshape(120, 16, 5, 5).transpose(2, 1, 3, 0).reshape(5, 80, 120)
    bfc1 = fc1_b[None, :]
    wfc2 = fc2_w.T
    bfc2 = fc2_b[None, :]
    gd = (gate_w[:, 1] - gate_w[:, 0])[:, None]
    we = expert_w.transpose(0, 2, 1)
    wf4 = fc4_w.T
    bf4 = fc4_b[None, :]

    grid = (b // nb,)
    const = lambda *s: pl.BlockSpec(s, lambda i: (0,) * len(s))
    return pl.pallas_call(
        functools.partial(_net_kernel, nb),
        grid=grid,
        in_specs=[
            pl.BlockSpec((nb, 32, 96), lambda i: (i, 0, 0)),
            const(5, 96, 84), const(5, 96, 84), const(1, 84),
            const(5, 84, 80), const(5, 84, 80), const(1, 80),
            const(5, 80, 120), const(1, 120),
            const(120, 84), const(1, 84),
            const(84, 1),
            const(2, 84, 84), const(2, 84),
            const(84, 10), const(1, 10),
        ],
        out_specs=pl.BlockSpec((nb, 10), lambda i: (i, 0)),
        out_shape=jax.ShapeDtypeStruct((b, 10), jnp.float32),
    )(xt, w1e, w1o, b1, w2e, w2o, b2, wfc1, bfc1, wfc2, bfc2, gd, we,
      expert_b, wf4, bf4)


# speedup vs baseline: 3.4473x; 1.4823x over previous
import functools

import jax
import jax.numpy as jnp
from jax.experimental import pallas as pl


def _toeplitz_conv1(conv1_w, parity):
    iw = jnp.arange(32)[:, None]
    j = jnp.arange(14)[None, :]
    kx = iw - (2 * j + parity)
    mask = (kx >= 0) & (kx < 5)
    g = conv1_w[:, :, :, jnp.clip(kx, 0, 4)]
    g = g * mask[None, None, None, :, :]
    return g.transpose(2, 1, 3, 0, 4).reshape(5, 96, 84)


def _toeplitz_conv2(conv2_w, parity):
    iw = jnp.arange(14)[:, None]
    j = jnp.arange(5)[None, :]
    kx = iw - (2 * j + parity)
    mask = (kx >= 0) & (kx < 5)
    g = conv2_w[:, :, :, jnp.clip(kx, 0, 4)]
    g = g * mask[None, None, None, :, :]
    return g.transpose(2, 1, 3, 0, 4).reshape(5, 84, 80)


def _net_kernel(nb, xt_ref, w1e_ref, w1o_ref, b1_ref, w2e_ref, w2o_ref,
                b2_ref, wfc1_ref, bfc1_ref, wfc2_ref, bfc2_ref, gd_ref,
                we_ref, be_ref, wf4_ref, bf4_ref, out_ref):
    f32 = jnp.float32
    x = xt_ref[...]

    acc_e = None
    acc_o = None
    for ky in range(5):
        a = x[ky:ky + 28].reshape(28 * nb, 96)
        pe = jnp.dot(a, w1e_ref[ky], preferred_element_type=f32)
        po = jnp.dot(a, w1o_ref[ky], preferred_element_type=f32)
        acc_e = pe if acc_e is None else acc_e + pe
        acc_o = po if acc_o is None else acc_o + po
    t = jnp.maximum(acc_e, acc_o) + b1_ref[...]
    t = t.reshape(14, 2, nb, 84).max(axis=1)
    h1 = jnp.maximum(t, 0.0)

    acc_e = None
    acc_o = None
    for ky in range(5):
        a = h1[ky:ky + 10].reshape(10 * nb, 84)
        pe = jnp.dot(a, w2e_ref[ky], preferred_element_type=f32)
        po = jnp.dot(a, w2o_ref[ky], preferred_element_type=f32)
        acc_e = pe if acc_e is None else acc_e + pe
        acc_o = po if acc_o is None else acc_o + po
    t2 = jnp.maximum(acc_e, acc_o) + b2_ref[...]
    t2 = t2.reshape(5, 2, nb, 80).max(axis=1)
    h2 = jnp.maximum(t2, 0.0)

    acc3 = None
    for y in range(5):
        p = jnp.dot(h2[y], wfc1_ref[y], preferred_element_type=f32)
        acc3 = p if acc3 is None else acc3 + p
    h3 = jnp.maximum(acc3 + bfc1_ref[...], 0.0)

    h4 = jnp.dot(h3, wfc2_ref[...], preferred_element_type=f32)
    h4 = jnp.maximum(h4 + bfc2_ref[...], 0.0)

    dlog = jnp.dot(h4, gd_ref[...], preferred_element_type=f32)
    gate_val = jax.nn.sigmoid(jnp.abs(dlog))
    sel = dlog > 0.0

    eo0 = jnp.dot(h4, we_ref[0], preferred_element_type=f32) + be_ref[0:1, :]
    eo1 = jnp.dot(h4, we_ref[1], preferred_element_type=f32) + be_ref[1:2, :]
    eo = jnp.where(sel, eo1, eo0) * gate_val

    out_ref[...] = jnp.dot(eo, wf4_ref[...], preferred_element_type=f32) \
        + bf4_ref[...]


def kernel(x, conv1_w, conv1_b, conv2_w, conv2_b, fc1_w, fc1_b, fc2_w,
           fc2_b, gate_w, expert_w, expert_b, fc4_w, fc4_b):
    b = x.shape[0]
    nb = 256 if b % 256 == 0 else b

    xt = x.transpose(2, 0, 1, 3).reshape(32, b, 96)

    w1e = _toeplitz_conv1(conv1_w, 0)
    w1o = _toeplitz_conv1(conv1_w, 1)
    b1 = jnp.repeat(conv1_b, 14)[None, :]
    w2e = _toeplitz_conv2(conv2_w, 0)
    w2o = _toeplitz_conv2(conv2_w, 1)
    b2 = jnp.repeat(conv2_b, 5)[None, :]
    wfc1 = fc1_w.reshape(120, 16, 5, 5).transpose(2, 1, 3, 0).reshape(5, 80, 120)
    bfc1 = fc1_b[None, :]
    wfc2 = fc2_w.T
    bfc2 = fc2_b[None, :]
    gd = (gate_w[:, 1] - gate_w[:, 0])[:, None]
    we = expert_w.transpose(0, 2, 1)
    wf4 = fc4_w.T
    bf4 = fc4_b[None, :]

    grid = (b // nb,)
    const = lambda *s: pl.BlockSpec(s, lambda i: (0,) * len(s))
    return pl.pallas_call(
        functools.partial(_net_kernel, nb),
        grid=grid,
        in_specs=[
            pl.BlockSpec((32, nb, 96), lambda i: (0, i, 0)),
            const(5, 96, 84), const(5, 96, 84), const(1, 84),
            const(5, 84, 80), const(5, 84, 80), const(1, 80),
            const(5, 80, 120), const(1, 120),
            const(120, 84), const(1, 84),
            const(84, 1),
            const(2, 84, 84), const(2, 84),
            const(84, 10), const(1, 10),
        ],
        out_specs=pl.BlockSpec((nb, 10), lambda i: (i, 0)),
        out_shape=jax.ShapeDtypeStruct((b, 10), jnp.float32),
    )(xt, w1e, w1o, b1, w2e, w2o, b2, wfc1, bfc1, wfc2, bfc2, gd, we,
      expert_b, wf4, bf4)
